# baseline (speedup 1.0000x reference)
import functools

import jax
import jax.numpy as jnp
from jax import lax
from jax.experimental import pallas as pl
from jax.experimental.pallas import tpu as pltpu
from jax.experimental.pallas import tpu_sc as plsc

B, C, I, N = 256, 64, 64, 32
ROW = N * 2
NW = 32
B_PER_W = B // NW
L = 16
CV = C // L


def _matches_and_gather(instf_hbm, inst_hbm, cent_hbm, out_hbm, inst_v,
                        cent_v, idx_v, rows_v, sem):
    wid = lax.axis_index("s") * 2 + lax.axis_index("c")
    lanes = lax.iota(jnp.int32, L)
    zero_l = jnp.zeros((L,), jnp.int32)
    inf_l = jnp.full((L,), jnp.inf, jnp.float32)
    one_l = jnp.full((L,), 1, jnp.int32)
    row_l = jnp.full((L,), ROW, jnp.int32)

    def do_batch(b):
        pltpu.sync_copy(instf_hbm.at[pl.ds(b * I * ROW, I * ROW)], inst_v)
        pltpu.sync_copy(cent_hbm.at[pl.ds(b * C * 2, C * 2)], cent_v)
        cx = [plsc.load_gather(cent_v, [2 * (lanes + L * j)])
              for j in range(CV)]
        cy = [plsc.load_gather(cent_v, [2 * (lanes + L * j) + 1])
              for j in range(CV)]

        def i_body(i, carry):
            best, besti, irow, ispl = carry
            dmin = [inf_l] * CV
            for n in range(N):
                ax = plsc.load_gather(inst_v, [irow + (2 * n)])
                ay = plsc.load_gather(inst_v, [irow + (2 * n + 1)])
                for j in range(CV):
                    dx = ax - cx[j]
                    dy = ay - cy[j]
                    d = dx * dx + dy * dy
                    dmin[j] = jnp.minimum(dmin[j], d)
            newb, newi = [], []
            for j in range(CV):
                better = dmin[j] < best[j]
                newb.append(jnp.where(better, dmin[j], best[j]))
                newi.append(jnp.where(better, ispl, besti[j]))
            return (tuple(newb), tuple(newi), irow + row_l, ispl + one_l)

        init = (tuple([inf_l] * CV), tuple([zero_l] * CV), zero_l, zero_l)
        _, besti, _, _ = lax.fori_loop(0, I, i_body, init)

        b64 = jnp.full((L,), b * I, jnp.int32)
        for j in range(CV):
            idx_v[pl.ds(j * L, L)] = besti[j] + b64
        pltpu.async_copy(inst_hbm.at[idx_v], rows_v, sem).wait()
        pltpu.sync_copy(rows_v, out_hbm.at[pl.ds(b * C, C)])

    def batch_body(bi, _):
        do_batch(wid * B_PER_W + bi)
        return 0

    lax.fori_loop(0, B_PER_W, batch_body, 0)


@jax.jit
def kernel(instances, centroids, centroid_vals):
    inst_flat = instances.reshape(B * I, ROW)
    inst_1d = instances.reshape(B * I * ROW)
    cent_1d = centroids.reshape(B * C * 2)
    mesh = plsc.VectorSubcoreMesh(core_axis_name="c", subcore_axis_name="s")
    k = functools.partial(
        pl.kernel,
        mesh=mesh,
        compiler_params=pltpu.CompilerParams(
            needs_layout_passes=False, use_tc_tiling_on_sc=False),
        out_type=jax.ShapeDtypeStruct((B * C, ROW), jnp.float32),
        scratch_types=[
            pltpu.VMEM((I * ROW,), jnp.float32),
            pltpu.VMEM((C * 2,), jnp.float32),
            pltpu.VMEM((C,), jnp.int32),
            pltpu.VMEM((C, ROW), jnp.float32),
            pltpu.SemaphoreType.DMA,
        ],
    )(_matches_and_gather)
    peaks_flat = k(inst_1d, inst_flat, cent_1d)
    instance_peaks = peaks_flat.reshape(B, C, N, 2)
    instance_peak_vals = jnp.ones((B, C, N), jnp.float32)
    return (centroids, centroid_vals, instance_peaks, instance_peak_vals)
